# Initial kernel scaffold; baseline (speedup 1.0000x reference)
import jax
import jax.numpy as jnp
from jax.experimental import pallas as pl


def kernel(atom_in_fea, nbr_fea, nbr_fea_idx, tabulated_padding_fillter, W_full, b_full, bn1_gamma, bn1_beta, bn2_gamma, bn2_beta):
    raise NotImplementedError("write your pallas kernel here")


# speedup vs baseline: 2.7541x; 2.7541x over previous
import functools

import jax
import jax.numpy as jnp
from jax import lax
from jax.experimental import pallas as pl
from jax.experimental.pallas import tpu as pltpu
from jax.experimental.pallas import tpu_sc as plsc

_NC = 2
_NS = 16
_EPS = 1e-5


def _make_gather(v, d, b, ch):
    nw = _NC * _NS
    b_per_w = b // nw
    n_ch = b_per_w // ch
    mesh = plsc.VectorSubcoreMesh(core_axis_name="c", subcore_axis_name="s")

    @functools.partial(
        pl.kernel,
        mesh=mesh,
        out_type=jax.ShapeDtypeStruct((b, d), jnp.float32),
        scratch_types=[
            pltpu.VMEM((ch,), jnp.int32),
            pltpu.VMEM((ch, d), jnp.float32),
            pltpu.SemaphoreType.DMA,
        ],
    )
    def gather_k(table_hbm, idx_hbm, out_hbm, idx_v, rows_v, sem):
        wid = lax.axis_index("s") * _NC + lax.axis_index("c")
        base = wid * b_per_w

        def step(c, carry):
            off = base + c * ch
            pltpu.sync_copy(idx_hbm.at[pl.ds(off, ch)], idx_v)
            pltpu.async_copy(table_hbm.at[idx_v], rows_v, sem).wait()
            pltpu.sync_copy(rows_v, out_hbm.at[pl.ds(off, ch)])
            return carry

        lax.fori_loop(0, n_ch, step, 0)

    return gather_k


def _stats_body(atom_ref, gath_ref, nbr_ref, mask_ref, wsT_ref, wnT_ref,
                weT_ref, b_ref, sum_ref, sq_ref, cnt_ref):
    i = pl.program_id(0)
    ba, mm = mask_ref.shape
    f = wsT_ref.shape[1]
    s = jnp.dot(atom_ref[...], wsT_ref[...],
                preferred_element_type=jnp.float32) + b_ref[...]
    g = jnp.dot(gath_ref[...], wnT_ref[...],
                preferred_element_type=jnp.float32)
    e = jnp.dot(nbr_ref[...], weT_ref[...],
                preferred_element_type=jnp.float32)
    x3 = (g + e).reshape(ba, mm, f) + s[:, None, :]
    m3 = mask_ref[...][:, :, None]
    xm = x3 * m3
    sx = jnp.sum(jnp.sum(xm, axis=1), axis=0)[None, :]
    sq = jnp.sum(jnp.sum(xm * x3, axis=1), axis=0)[None, :]
    c = jnp.sum(mask_ref[...])

    @pl.when(i == 0)
    def _():
        sum_ref[...] = jnp.zeros_like(sum_ref)
        sq_ref[...] = jnp.zeros_like(sq_ref)
        cnt_ref[...] = jnp.zeros_like(cnt_ref)

    sum_ref[...] += sx
    sq_ref[...] += sq
    cnt_ref[...] = cnt_ref[...] + c


def _apply_body(atom_ref, gath_ref, nbr_ref, mask_ref, wsT_ref, wnT_ref,
                weT_ref, b_ref, sum_ref, sq_ref, cnt_ref, g1_ref, b1_ref,
                ns_ref, s2_ref, q2_ref):
    i = pl.program_id(0)
    ba, mm = mask_ref.shape
    f = wsT_ref.shape[1]
    d = f // 2
    s = jnp.dot(atom_ref[...], wsT_ref[...],
                preferred_element_type=jnp.float32) + b_ref[...]
    g = jnp.dot(gath_ref[...], wnT_ref[...],
                preferred_element_type=jnp.float32)
    e = jnp.dot(nbr_ref[...], weT_ref[...],
                preferred_element_type=jnp.float32)
    x3 = (g + e).reshape(ba, mm, f) + s[:, None, :]
    cnt = cnt_ref[0, 0]
    mean = sum_ref[...] / cnt
    var = sq_ref[...] / cnt - mean * mean
    scale = lax.rsqrt(var + _EPS) * g1_ref[...]
    shift = b1_ref[...] - mean * scale
    xn3 = x3 * scale[None, :, :] + shift[None, :, :]
    filt = jax.nn.sigmoid(xn3[:, :, :d])
    core = jax.nn.softplus(xn3[:, :, d:])
    prod = filt * core * mask_ref[...][:, :, None]
    ns = jnp.sum(prod, axis=1)
    ns_ref[...] = ns

    @pl.when(i == 0)
    def _():
        s2_ref[...] = jnp.zeros_like(s2_ref)
        q2_ref[...] = jnp.zeros_like(q2_ref)

    s2_ref[...] += jnp.sum(ns, axis=0)[None, :]
    q2_ref[...] += jnp.sum(ns * ns, axis=0)[None, :]


def _final_body(n_total, atom_ref, ns_ref, s2_ref, q2_ref, g2_ref, b2_ref,
                out_ref):
    mean2 = s2_ref[...] / n_total
    var2 = q2_ref[...] / n_total - mean2 * mean2
    scale = lax.rsqrt(var2 + _EPS) * g2_ref[...]
    shift = b2_ref[...] - mean2 * scale
    out_ref[...] = jax.nn.softplus(
        atom_ref[...] + ns_ref[...] * scale + shift)


def _stats_kwargs(n, m, d, f, e_dim, ba):
    nb = n // ba
    bspec = pl.BlockSpec
    return dict(
        grid=(nb,),
        in_specs=[
            bspec((ba, d), lambda i: (i, 0)),
            bspec((ba * m, d), lambda i: (i, 0)),
            bspec((ba * m, e_dim), lambda i: (i, 0)),
            bspec((ba, m), lambda i: (i, 0)),
            bspec((d, f), lambda i: (0, 0)),
            bspec((d, f), lambda i: (0, 0)),
            bspec((e_dim, f), lambda i: (0, 0)),
            bspec((1, f), lambda i: (0, 0)),
        ],
        out_specs=[
            bspec((1, f), lambda i: (0, 0)),
            bspec((1, f), lambda i: (0, 0)),
            bspec((1, 1), lambda i: (0, 0)),
        ],
        out_shape=[
            jax.ShapeDtypeStruct((1, f), jnp.float32),
            jax.ShapeDtypeStruct((1, f), jnp.float32),
            jax.ShapeDtypeStruct((1, 1), jnp.float32),
        ],
    )


def _apply_kwargs(n, m, d, f, e_dim, ba):
    nb = n // ba
    bspec = pl.BlockSpec
    return dict(
        grid=(nb,),
        in_specs=[
            bspec((ba, d), lambda i: (i, 0)),
            bspec((ba * m, d), lambda i: (i, 0)),
            bspec((ba * m, e_dim), lambda i: (i, 0)),
            bspec((ba, m), lambda i: (i, 0)),
            bspec((d, f), lambda i: (0, 0)),
            bspec((d, f), lambda i: (0, 0)),
            bspec((e_dim, f), lambda i: (0, 0)),
            bspec((1, f), lambda i: (0, 0)),
            bspec((1, f), lambda i: (0, 0)),
            bspec((1, f), lambda i: (0, 0)),
            bspec((1, 1), lambda i: (0, 0)),
            bspec((1, f), lambda i: (0, 0)),
            bspec((1, f), lambda i: (0, 0)),
        ],
        out_specs=[
            bspec((ba, d), lambda i: (i, 0)),
            bspec((1, d), lambda i: (0, 0)),
            bspec((1, d), lambda i: (0, 0)),
        ],
        out_shape=[
            jax.ShapeDtypeStruct((n, d), jnp.float32),
            jax.ShapeDtypeStruct((1, d), jnp.float32),
            jax.ShapeDtypeStruct((1, d), jnp.float32),
        ],
    )


def _final_kwargs(n, d, ba):
    nb = n // ba
    bspec = pl.BlockSpec
    return dict(
        grid=(nb,),
        in_specs=[
            bspec((ba, d), lambda i: (i, 0)),
            bspec((ba, d), lambda i: (i, 0)),
            bspec((1, d), lambda i: (0, 0)),
            bspec((1, d), lambda i: (0, 0)),
            bspec((1, d), lambda i: (0, 0)),
            bspec((1, d), lambda i: (0, 0)),
        ],
        out_specs=bspec((ba, d), lambda i: (i, 0)),
        out_shape=jax.ShapeDtypeStruct((n, d), jnp.float32),
    )


def kernel(atom_in_fea, nbr_fea, nbr_fea_idx, tabulated_padding_fillter,
           W_full, b_full, bn1_gamma, bn1_beta, bn2_gamma, bn2_beta):
    n, m = nbr_fea_idx.shape
    d = atom_in_fea.shape[1]
    e_dim = nbr_fea.shape[2]
    f = 2 * d
    ba = 200

    idx = nbr_fea_idx.astype(jnp.int32).reshape(-1)
    gathered = _make_gather(n, d, n * m, 400)(atom_in_fea, idx)

    wsT = W_full[:, :d].T
    wnT = W_full[:, d:2 * d].T
    weT = W_full[:, 2 * d:].T
    nbr2 = nbr_fea.reshape(n * m, e_dim)
    mask = tabulated_padding_fillter
    b2 = b_full.reshape(1, f)
    g1 = bn1_gamma.reshape(1, f)
    be1 = bn1_beta.reshape(1, f)
    g2 = bn2_gamma.reshape(1, d)
    be2 = bn2_beta.reshape(1, d)

    sum1, sq1, cnt = pl.pallas_call(
        _stats_body, **_stats_kwargs(n, m, d, f, e_dim, ba))(
        atom_in_fea, gathered, nbr2, mask, wsT, wnT, weT, b2)

    ns, s2, q2 = pl.pallas_call(
        _apply_body, **_apply_kwargs(n, m, d, f, e_dim, ba))(
        atom_in_fea, gathered, nbr2, mask, wsT, wnT, weT, b2,
        sum1, sq1, cnt, g1, be1)

    out = pl.pallas_call(
        functools.partial(_final_body, float(n)),
        **_final_kwargs(n, d, ba))(
        atom_in_fea, ns, s2, q2, g2, be2)
    return out
